# Initial kernel scaffold; baseline (speedup 1.0000x reference)
import jax
import jax.numpy as jnp
from jax.experimental import pallas as pl


def kernel(q_sub, q_rel, hidden, edges, nodes, id_layer, n_layer, old_nodes_new_idx, rel_table, gru_Wi, gru_Wh, gru_bi, gru_bh, pna_W, pna_b, pr_W1, pr_b1, pr_W2, pr_b2):
    raise NotImplementedError("write your pallas kernel here")


# speedup vs baseline: 1.2547x; 1.2547x over previous
import functools

import jax
import jax.numpy as jnp
from jax import lax
from jax.experimental import pallas as pl
from jax.experimental.pallas import tpu as pltpu

D = 128
N_USER = 1000
BATCH = 8
DELTA = 2.5
TAU = 1.0


def _msg_body(hs_ref, rel_ref, wh_ref, bh_ref, relgi_ref, out_ref):
    hs = hs_ref[...]
    gh = jnp.dot(hs, wh_ref[...], preferred_element_type=jnp.float32)
    gh = gh + bh_ref[...]
    rel = rel_ref[0, 0, :]
    nrel = relgi_ref.shape[0]
    oh = (rel[:, None] == lax.broadcasted_iota(jnp.int32, (rel.shape[0], nrel), 1)
          ).astype(jnp.float32)
    gi = jnp.dot(oh, relgi_ref[...], preferred_element_type=jnp.float32)
    d = hs.shape[1]
    i_r, i_z, i_n = gi[:, :d], gi[:, d:2 * d], gi[:, 2 * d:]
    h_r, h_z, h_n = gh[:, :d], gh[:, d:2 * d], gh[:, 2 * d:]
    r = jax.nn.sigmoid(i_r + h_r)
    z = jax.nn.sigmoid(i_z + h_z)
    n = jnp.tanh(i_n + r * h_n)
    out_ref[...] = (1.0 - z) * n + z * hs


def _compute_messages(hs_g, rel, gru_Wh, gru_bh, rel_gi, eb=2000):
    e = hs_g.shape[0]
    d = hs_g.shape[1]
    nrel = rel_gi.shape[0]
    nb = e // eb
    rel3 = rel.reshape(nb, 1, eb)
    return pl.pallas_call(
        _msg_body,
        grid=(nb,),
        in_specs=[
            pl.BlockSpec((eb, d), lambda i: (i, 0)),
            pl.BlockSpec((1, 1, eb), lambda i: (i, 0, 0)),
            pl.BlockSpec((d, 3 * d), lambda i: (0, 0)),
            pl.BlockSpec((1, 3 * d), lambda i: (0, 0)),
            pl.BlockSpec((nrel, 3 * d), lambda i: (0, 0)),
        ],
        out_specs=pl.BlockSpec((eb, d), lambda i: (i, 0)),
        out_shape=jax.ShapeDtypeStruct((e, d), jnp.float32),
    )(hs_g, rel3, gru_Wh, gru_bh.reshape(1, 3 * d), rel_gi)


def _pna_body(ssum_ref, sq_ref, mx_ref, mn_ref, deg_ref, cnt01_ref,
              hpn_ref, nodes_ref, pnaw_ref, pnab_ref,
              ht_ref, usum_ref, ucnt_ref):
    i = pl.program_id(0)
    deg = deg_ref[...]
    inv = 1.0 / deg
    cnt01 = cnt01_ref[...]
    mean = ssum_ref[...] * inv
    var = jnp.maximum(sq_ref[...] * inv - mean * mean, 0.0)
    std = jnp.sqrt(var + 1e-6)
    mx = jnp.where(cnt01 > 0.0, mx_ref[...], 0.0)
    mn = jnp.where(cnt01 > 0.0, mn_ref[...], 0.0)
    aggs = jnp.concatenate([mean, mx, mn, std], axis=1)
    logd = jnp.log(deg + 1.0)
    s_amp = logd * (1.0 / DELTA)
    s_att = DELTA / logd
    feat = jnp.concatenate(
        [hpn_ref[...], aggs, aggs * s_amp, aggs * s_att], axis=1)
    ht = jnp.dot(feat, pnaw_ref[...], preferred_element_type=jnp.float32)
    ht = ht + pnab_ref[...]
    ht_ref[...] = ht

    nodes = nodes_ref[...]
    n_batch = nodes[:, 0]
    n_id = nodes[:, 1]
    uw = (n_id < N_USER).astype(jnp.float32)
    oh = (n_batch[:, None] == lax.broadcasted_iota(
        jnp.int32, (nodes.shape[0], BATCH), 1)).astype(jnp.float32)
    hw = ht * uw[:, None]
    usum_part = lax.dot_general(oh, hw, (((0,), (0,)), ((), ())),
                                preferred_element_type=jnp.float32)
    uw2 = jnp.broadcast_to(uw[:, None], ht.shape)
    ucnt_part = lax.dot_general(oh, uw2, (((0,), (0,)), ((), ())),
                                preferred_element_type=jnp.float32)

    @pl.when(i == 0)
    def _():
        usum_ref[...] = jnp.zeros_like(usum_ref)
        ucnt_ref[...] = jnp.zeros_like(ucnt_ref)

    usum_ref[...] += usum_part
    ucnt_ref[...] += ucnt_part


def _pna(ssum, sq, mx, mn, deg, cnt01, hpn, nodes, pna_W, pna_b, nbk=1000):
    n, d = ssum.shape
    grid = (n // nbk,)
    blk = lambda w: pl.BlockSpec((nbk, w), lambda i: (i, 0))
    fix = lambda s: pl.BlockSpec(s, lambda i: tuple(0 for _ in s))
    return pl.pallas_call(
        _pna_body,
        grid=grid,
        in_specs=[blk(d), blk(d), blk(d), blk(d), blk(1), blk(1), blk(d),
                  blk(2), fix((13 * d, d)), fix((1, d))],
        out_specs=[blk(d), fix((BATCH, d)), fix((BATCH, d))],
        out_shape=[jax.ShapeDtypeStruct((n, d), jnp.float32),
                   jax.ShapeDtypeStruct((BATCH, d), jnp.float32),
                   jax.ShapeDtypeStruct((BATCH, d), jnp.float32)],
    )(ssum, sq, mx, mn, deg, cnt01, hpn, nodes, pna_W, pna_b.reshape(1, d))


def _prune_body(ht_ref, nodes_ref, huser_ref, w1_ref, b1_ref, w2_ref, b2_ref,
                hg_ref, alpha_ref):
    ht = ht_ref[...]
    nodes = nodes_ref[...]
    n_batch = nodes[:, 0]
    oh = (n_batch[:, None] == lax.broadcasted_iota(
        jnp.int32, (nodes.shape[0], BATCH), 1)).astype(jnp.float32)
    hu = jnp.dot(oh, huser_ref[...], preferred_element_type=jnp.float32)
    feat = jnp.concatenate([hu, ht], axis=1)
    l1 = jnp.maximum(
        jnp.dot(feat, w1_ref[...], preferred_element_type=jnp.float32)
        + b1_ref[...], 0.0)
    logit = jnp.dot(l1, w2_ref[...], preferred_element_type=jnp.float32)
    logit = logit + b2_ref[...]
    alpha = jax.nn.sigmoid(logit * (1.0 / TAU))
    alpha_ref[...] = alpha
    hg_ref[...] = alpha * ht


def _prune(ht, nodes, h_user, pr_W1, pr_b1, pr_W2, pr_b2, nbk=1000):
    n, d = ht.shape
    hid = pr_W1.shape[1]
    w2p = jnp.broadcast_to(pr_W2, (hid, d))
    b2p = jnp.broadcast_to(pr_b2.reshape(1, 1), (1, d))
    grid = (n // nbk,)
    blk = lambda w: pl.BlockSpec((nbk, w), lambda i: (i, 0))
    fix = lambda s: pl.BlockSpec(s, lambda i: tuple(0 for _ in s))
    return pl.pallas_call(
        _prune_body,
        grid=grid,
        in_specs=[blk(d), blk(2), fix((BATCH, d)), fix((2 * d, hid)),
                  fix((1, hid)), fix((hid, d)), fix((1, d))],
        out_specs=[blk(d), blk(d)],
        out_shape=[jax.ShapeDtypeStruct((n, d), jnp.float32),
                   jax.ShapeDtypeStruct((n, d), jnp.float32)],
    )(ht, nodes, h_user, pr_W1, pr_b1.reshape(1, hid), w2p, b2p)


def kernel(q_sub, q_rel, hidden, edges, nodes, id_layer, n_layer,
           old_nodes_new_idx, rel_table, gru_Wi, gru_Wh, gru_bi, gru_bh,
           pna_W, pna_b, pr_W1, pr_b1, pr_W2, pr_b2):
    n, d = hidden.shape
    e = edges.shape[0]

    sub = edges[:, 4]
    rel = edges[:, 2]
    obj = edges[:, 5]

    rel_gi = rel_table @ gru_Wi + gru_bi

    hs_g = hidden[sub]
    messages = _compute_messages(hs_g, rel, gru_Wh, gru_bh, rel_gi)

    cnt = jax.ops.segment_sum(jnp.ones((e,), jnp.float32), obj, num_segments=n)
    ssum = jax.ops.segment_sum(messages, obj, num_segments=n)
    mx = jax.ops.segment_max(messages, obj, num_segments=n)
    mn = -jax.ops.segment_max(-messages, obj, num_segments=n)
    sq = jax.ops.segment_sum(messages * messages, obj, num_segments=n)

    deg = jnp.maximum(cnt, 1.0).reshape(n, 1)
    cnt01 = (cnt > 0).astype(jnp.float32).reshape(n, 1)
    mx = jnp.where(cnt01 > 0, mx, 0.0)
    mn = jnp.where(cnt01 > 0, mn, 0.0)

    hpn = jnp.zeros((n, d), jnp.float32).at[old_nodes_new_idx].set(hidden)

    ht, usum, ucnt = _pna(ssum, sq, mx, mn, deg, cnt01, hpn, nodes,
                          pna_W, pna_b)
    h_user = usum / jnp.maximum(ucnt, 1.0)

    h_gated, alpha2 = _prune(ht, nodes, h_user, pr_W1, pr_b1, pr_W2, pr_b2)
    alpha = alpha2[:, 0]

    sampled_nodes_idx = jnp.ones((n,), dtype=bool)
    final_nodes = jnp.array([0], dtype=jnp.int32)
    return (h_gated, nodes, final_nodes, old_nodes_new_idx,
            sampled_nodes_idx, alpha, edges)


# speedup vs baseline: 1.4619x; 1.1651x over previous
import functools

import jax
import jax.numpy as jnp
from jax import lax
from jax.experimental import pallas as pl
from jax.experimental.pallas import tpu as pltpu
from jax.experimental.pallas import tpu_sc as plsc

D = 128
N_USER = 1000
BATCH = 8
DELTA = 2.5
TAU = 1.0

SC_CORES = 2
SC_SUBCORES = 16
NW = SC_CORES * SC_SUBCORES


def _sc_gather_scatter(hidden, sub, perm):
    n, d = hidden.shape
    e = sub.shape[0]
    per_w = e // NW
    ch = 128
    nch = per_w // ch
    rem = per_w - nch * ch
    sper = (n // NW) // 8 * 8
    srem = n - sper * NW
    sch = 104
    nsch = sper // sch
    assert nsch * sch == sper and rem % 8 == 0 and srem % 8 == 0
    assert rem == srem and rem >= 8

    mesh = plsc.VectorSubcoreMesh(core_axis_name="c", subcore_axis_name="s")

    @functools.partial(
        pl.kernel,
        out_type=[jax.ShapeDtypeStruct((e, d), jnp.float32),
                  jax.ShapeDtypeStruct((n, d), jnp.float32)],
        mesh=mesh,
        scratch_types=[
            pltpu.VMEM((ch,), jnp.int32),
            pltpu.VMEM((ch, d), jnp.float32),
            pltpu.VMEM((max(rem, 8),), jnp.int32),
            pltpu.VMEM((max(rem, 8), d), jnp.float32),
            pltpu.VMEM((sch,), jnp.int32),
            pltpu.SemaphoreType.DMA,
        ],
    )
    def k(hidden_hbm, sub_hbm, perm_hbm, out_hbm, hpn_hbm,
          idx_v, rows_v, idx_r, rows_r, idx_s, sem):
        wid = lax.axis_index("s") * SC_CORES + lax.axis_index("c")
        base = wid * per_w

        def gather_chunk(c, _):
            off = base + c * ch
            pltpu.sync_copy(sub_hbm.at[pl.ds(off, ch)], idx_v)
            pltpu.async_copy(hidden_hbm.at[idx_v], rows_v, sem).wait()
            pltpu.sync_copy(rows_v, out_hbm.at[pl.ds(off, ch)])
            return _

        lax.fori_loop(0, nch, gather_chunk, None)
        if rem:
            off = base + nch * ch
            pltpu.sync_copy(sub_hbm.at[pl.ds(off, rem)], idx_r)
            pltpu.async_copy(hidden_hbm.at[idx_r], rows_r, sem).wait()
            pltpu.sync_copy(rows_r, out_hbm.at[pl.ds(off, rem)])

        sbase = wid * sper

        def scatter_chunk(c, _):
            off = sbase + c * sch
            pltpu.sync_copy(perm_hbm.at[pl.ds(off, sch)], idx_s)
            pltpu.sync_copy(hidden_hbm.at[pl.ds(off, sch)],
                            rows_v.at[pl.ds(0, sch)])
            pltpu.async_copy(rows_v.at[pl.ds(0, sch)],
                             hpn_hbm.at[idx_s], sem).wait()
            return _

        lax.fori_loop(0, nsch, scatter_chunk, None)

        @pl.when(wid == NW - 1)
        def _():
            off = NW * sper
            pltpu.sync_copy(perm_hbm.at[pl.ds(off, srem)], idx_r)
            pltpu.sync_copy(hidden_hbm.at[pl.ds(off, srem)], rows_r)
            pltpu.async_copy(rows_r, hpn_hbm.at[idx_r], sem).wait()

    return k(hidden, sub, perm)


def _msg_body(hs_ref, rel_ref, wh_ref, bh_ref, relgi_ref, out_ref):
    hs = hs_ref[...]
    gh = jnp.dot(hs, wh_ref[...], preferred_element_type=jnp.float32)
    gh = gh + bh_ref[...]
    rel = rel_ref[0, 0, :]
    nrel = relgi_ref.shape[0]
    oh = (rel[:, None] == lax.broadcasted_iota(jnp.int32, (rel.shape[0], nrel), 1)
          ).astype(jnp.float32)
    gi = jnp.dot(oh, relgi_ref[...], preferred_element_type=jnp.float32)
    d = hs.shape[1]
    i_r, i_z, i_n = gi[:, :d], gi[:, d:2 * d], gi[:, 2 * d:]
    h_r, h_z, h_n = gh[:, :d], gh[:, d:2 * d], gh[:, 2 * d:]
    r = jax.nn.sigmoid(i_r + h_r)
    z = jax.nn.sigmoid(i_z + h_z)
    n = jnp.tanh(i_n + r * h_n)
    out_ref[...] = (1.0 - z) * n + z * hs


def _compute_messages(hs_g, rel, gru_Wh, gru_bh, rel_gi, eb=2000):
    e = hs_g.shape[0]
    d = hs_g.shape[1]
    nrel = rel_gi.shape[0]
    nb = e // eb
    rel3 = rel.reshape(nb, 1, eb)
    return pl.pallas_call(
        _msg_body,
        grid=(nb,),
        in_specs=[
            pl.BlockSpec((eb, d), lambda i: (i, 0)),
            pl.BlockSpec((1, 1, eb), lambda i: (i, 0, 0)),
            pl.BlockSpec((d, 3 * d), lambda i: (0, 0)),
            pl.BlockSpec((1, 3 * d), lambda i: (0, 0)),
            pl.BlockSpec((nrel, 3 * d), lambda i: (0, 0)),
        ],
        out_specs=pl.BlockSpec((eb, d), lambda i: (i, 0)),
        out_shape=jax.ShapeDtypeStruct((e, d), jnp.float32),
    )(hs_g, rel3, gru_Wh, gru_bh.reshape(1, 3 * d), rel_gi)


def _pna_body(ssum_ref, sq_ref, mx_ref, mn_ref, deg_ref, cnt01_ref,
              hpn_ref, nodes_ref, pnaw_ref, pnab_ref,
              ht_ref, usum_ref, ucnt_ref):
    i = pl.program_id(0)
    deg = deg_ref[...]
    inv = 1.0 / deg
    cnt01 = cnt01_ref[...]
    mean = ssum_ref[...] * inv
    var = jnp.maximum(sq_ref[...] * inv - mean * mean, 0.0)
    std = jnp.sqrt(var + 1e-6)
    mx = jnp.where(cnt01 > 0.0, mx_ref[...], 0.0)
    mn = jnp.where(cnt01 > 0.0, mn_ref[...], 0.0)
    aggs = jnp.concatenate([mean, mx, mn, std], axis=1)
    logd = jnp.log(deg + 1.0)
    s_amp = logd * (1.0 / DELTA)
    s_att = DELTA / logd
    feat = jnp.concatenate(
        [hpn_ref[...], aggs, aggs * s_amp, aggs * s_att], axis=1)
    ht = jnp.dot(feat, pnaw_ref[...], preferred_element_type=jnp.float32)
    ht = ht + pnab_ref[...]
    ht_ref[...] = ht

    nodes = nodes_ref[...]
    n_batch = nodes[:, 0]
    n_id = nodes[:, 1]
    uw = (n_id < N_USER).astype(jnp.float32)
    oh = (n_batch[:, None] == lax.broadcasted_iota(
        jnp.int32, (nodes.shape[0], BATCH), 1)).astype(jnp.float32)
    hw = ht * uw[:, None]
    usum_part = lax.dot_general(oh, hw, (((0,), (0,)), ((), ())),
                                preferred_element_type=jnp.float32)
    uw2 = jnp.broadcast_to(uw[:, None], ht.shape)
    ucnt_part = lax.dot_general(oh, uw2, (((0,), (0,)), ((), ())),
                                preferred_element_type=jnp.float32)

    @pl.when(i == 0)
    def _():
        usum_ref[...] = jnp.zeros_like(usum_ref)
        ucnt_ref[...] = jnp.zeros_like(ucnt_ref)

    usum_ref[...] += usum_part
    ucnt_ref[...] += ucnt_part


def _pna(ssum, sq, mx, mn, deg, cnt01, hpn, nodes, pna_W, pna_b, nbk=1000):
    n, d = ssum.shape
    grid = (n // nbk,)
    blk = lambda w: pl.BlockSpec((nbk, w), lambda i: (i, 0))
    fix = lambda s: pl.BlockSpec(s, lambda i: tuple(0 for _ in s))
    return pl.pallas_call(
        _pna_body,
        grid=grid,
        in_specs=[blk(d), blk(d), blk(d), blk(d), blk(1), blk(1), blk(d),
                  blk(2), fix((13 * d, d)), fix((1, d))],
        out_specs=[blk(d), fix((BATCH, d)), fix((BATCH, d))],
        out_shape=[jax.ShapeDtypeStruct((n, d), jnp.float32),
                   jax.ShapeDtypeStruct((BATCH, d), jnp.float32),
                   jax.ShapeDtypeStruct((BATCH, d), jnp.float32)],
    )(ssum, sq, mx, mn, deg, cnt01, hpn, nodes, pna_W, pna_b.reshape(1, d))


def _prune_body(ht_ref, nodes_ref, huser_ref, w1_ref, b1_ref, w2_ref, b2_ref,
                hg_ref, alpha_ref):
    ht = ht_ref[...]
    nodes = nodes_ref[...]
    n_batch = nodes[:, 0]
    oh = (n_batch[:, None] == lax.broadcasted_iota(
        jnp.int32, (nodes.shape[0], BATCH), 1)).astype(jnp.float32)
    hu = jnp.dot(oh, huser_ref[...], preferred_element_type=jnp.float32)
    feat = jnp.concatenate([hu, ht], axis=1)
    l1 = jnp.maximum(
        jnp.dot(feat, w1_ref[...], preferred_element_type=jnp.float32)
        + b1_ref[...], 0.0)
    logit = jnp.dot(l1, w2_ref[...], preferred_element_type=jnp.float32)
    logit = logit + b2_ref[...]
    alpha = jax.nn.sigmoid(logit * (1.0 / TAU))
    alpha_ref[...] = alpha
    hg_ref[...] = alpha * ht


def _prune(ht, nodes, h_user, pr_W1, pr_b1, pr_W2, pr_b2, nbk=1000):
    n, d = ht.shape
    hid = pr_W1.shape[1]
    w2p = jnp.broadcast_to(pr_W2, (hid, d))
    b2p = jnp.broadcast_to(pr_b2.reshape(1, 1), (1, d))
    grid = (n // nbk,)
    blk = lambda w: pl.BlockSpec((nbk, w), lambda i: (i, 0))
    fix = lambda s: pl.BlockSpec(s, lambda i: tuple(0 for _ in s))
    return pl.pallas_call(
        _prune_body,
        grid=grid,
        in_specs=[blk(d), blk(2), fix((BATCH, d)), fix((2 * d, hid)),
                  fix((1, hid)), fix((hid, d)), fix((1, d))],
        out_specs=[blk(d), blk(d)],
        out_shape=[jax.ShapeDtypeStruct((n, d), jnp.float32),
                   jax.ShapeDtypeStruct((n, d), jnp.float32)],
    )(ht, nodes, h_user, pr_W1, pr_b1.reshape(1, hid), w2p, b2p)


def kernel(q_sub, q_rel, hidden, edges, nodes, id_layer, n_layer,
           old_nodes_new_idx, rel_table, gru_Wi, gru_Wh, gru_bi, gru_bh,
           pna_W, pna_b, pr_W1, pr_b1, pr_W2, pr_b2):
    n, d = hidden.shape
    e = edges.shape[0]

    sub = edges[:, 4]
    rel = edges[:, 2]
    obj = edges[:, 5]

    rel_gi = rel_table @ gru_Wi + gru_bi

    sub_c = jnp.asarray(sub, jnp.int32)
    hs_g, hpn = _sc_gather_scatter(hidden, sub_c, old_nodes_new_idx)
    messages = _compute_messages(hs_g, rel, gru_Wh, gru_bh, rel_gi)

    cnt = jax.ops.segment_sum(jnp.ones((e,), jnp.float32), obj, num_segments=n)
    ssum = jax.ops.segment_sum(messages, obj, num_segments=n)
    mx = jax.ops.segment_max(messages, obj, num_segments=n)
    mn = -jax.ops.segment_max(-messages, obj, num_segments=n)
    sq = jax.ops.segment_sum(messages * messages, obj, num_segments=n)

    deg = jnp.maximum(cnt, 1.0).reshape(n, 1)
    cnt01 = (cnt > 0).astype(jnp.float32).reshape(n, 1)
    mx = jnp.where(cnt01 > 0, mx, 0.0)
    mn = jnp.where(cnt01 > 0, mn, 0.0)

    ht, usum, ucnt = _pna(ssum, sq, mx, mn, deg, cnt01, hpn, nodes,
                          pna_W, pna_b)
    h_user = usum / jnp.maximum(ucnt, 1.0)

    h_gated, alpha2 = _prune(ht, nodes, h_user, pr_W1, pr_b1, pr_W2, pr_b2)
    alpha = alpha2[:, 0]

    sampled_nodes_idx = jnp.ones((n,), dtype=bool)
    final_nodes = jnp.array([0], dtype=jnp.int32)
    return (h_gated, nodes, final_nodes, old_nodes_new_idx,
            sampled_nodes_idx, alpha, edges)


# speedup vs baseline: 1.9181x; 1.3121x over previous
import functools

import jax
import jax.numpy as jnp
from jax import lax
from jax.experimental import pallas as pl
from jax.experimental.pallas import tpu as pltpu
from jax.experimental.pallas import tpu_sc as plsc

D = 128
N_USER = 1000
BATCH = 8
DELTA = 2.5
TAU = 1.0

SC_CORES = 2
SC_SUBCORES = 16
NW = SC_CORES * SC_SUBCORES


def _sc_gather_scatter(hidden, sub, perm):
    n, d = hidden.shape
    e = sub.shape[0]
    per_w = e // NW
    ch = 128
    nch = per_w // ch
    rem = per_w - nch * ch
    sper = (n // NW) // 8 * 8
    srem = n - sper * NW
    sch = 104
    nsch = sper // sch
    assert nsch * sch == sper and rem % 8 == 0 and srem % 8 == 0
    assert rem == srem and rem >= 8

    mesh = plsc.VectorSubcoreMesh(core_axis_name="c", subcore_axis_name="s")

    @functools.partial(
        pl.kernel,
        out_type=[jax.ShapeDtypeStruct((e, d), jnp.float32),
                  jax.ShapeDtypeStruct((n, d), jnp.float32)],
        mesh=mesh,
        scratch_types=[
            pltpu.VMEM((ch,), jnp.int32),
            pltpu.VMEM((ch, d), jnp.float32),
            pltpu.VMEM((max(rem, 8),), jnp.int32),
            pltpu.VMEM((max(rem, 8), d), jnp.float32),
            pltpu.VMEM((sch,), jnp.int32),
            pltpu.SemaphoreType.DMA,
        ],
    )
    def k(hidden_hbm, sub_hbm, perm_hbm, out_hbm, hpn_hbm,
          idx_v, rows_v, idx_r, rows_r, idx_s, sem):
        wid = lax.axis_index("s") * SC_CORES + lax.axis_index("c")
        base = wid * per_w

        def gather_chunk(c, _):
            off = base + c * ch
            pltpu.sync_copy(sub_hbm.at[pl.ds(off, ch)], idx_v)
            pltpu.async_copy(hidden_hbm.at[idx_v], rows_v, sem).wait()
            pltpu.sync_copy(rows_v, out_hbm.at[pl.ds(off, ch)])
            return _

        lax.fori_loop(0, nch, gather_chunk, None)
        if rem:
            off = base + nch * ch
            pltpu.sync_copy(sub_hbm.at[pl.ds(off, rem)], idx_r)
            pltpu.async_copy(hidden_hbm.at[idx_r], rows_r, sem).wait()
            pltpu.sync_copy(rows_r, out_hbm.at[pl.ds(off, rem)])

        sbase = wid * sper

        def scatter_chunk(c, _):
            off = sbase + c * sch
            pltpu.sync_copy(perm_hbm.at[pl.ds(off, sch)], idx_s)
            pltpu.sync_copy(hidden_hbm.at[pl.ds(off, sch)],
                            rows_v.at[pl.ds(0, sch)])
            pltpu.async_copy(rows_v.at[pl.ds(0, sch)],
                             hpn_hbm.at[idx_s], sem).wait()
            return _

        lax.fori_loop(0, nsch, scatter_chunk, None)

        @pl.when(wid == NW - 1)
        def _():
            off = NW * sper
            pltpu.sync_copy(perm_hbm.at[pl.ds(off, srem)], idx_r)
            pltpu.sync_copy(hidden_hbm.at[pl.ds(off, srem)], rows_r)
            pltpu.async_copy(rows_r, hpn_hbm.at[idx_r], sem).wait()

    return k(hidden, sub, perm)


_SENT = 3.0e38


def _sc_segment_sums(msg, msgsq, obj):
    e, d = msg.shape
    n = 10000
    per_sc = e // SC_SUBCORES
    nch1 = per_sc // 128
    rem1 = per_sc - nch1 * 128
    zrows0 = 640
    assert rem1 % 8 == 0 and e % SC_SUBCORES == 0
    assert n - 15 * zrows0 == 400

    zblk = jnp.zeros((128, d), jnp.float32)

    mesh = plsc.VectorSubcoreMesh(core_axis_name="c", subcore_axis_name="s")

    @functools.partial(
        pl.kernel,
        out_type=[jax.ShapeDtypeStruct((n, d), jnp.float32),
                  jax.ShapeDtypeStruct((n, d), jnp.float32)],
        mesh=mesh,
        scratch_types=[
            pltpu.VMEM((128,), jnp.int32),
            pltpu.VMEM((128, d), jnp.float32),
            pltpu.VMEM((rem1,), jnp.int32),
            pltpu.VMEM((rem1, d), jnp.float32),
            pltpu.VMEM_SHARED((n, d), jnp.float32),
            pltpu.SemaphoreType.DMA,
        ],
    )
    def k(msg_hbm, sq_hbm, obj_hbm, zblk_hbm, ssum_hbm, ssq_hbm,
          idx_v, rows_v, idx_r, rows_r, acc_sh, sem):
        cid = lax.axis_index("c")
        sid = lax.axis_index("s")
        zbase = sid * zrows0

        @pl.when(sid < SC_SUBCORES - 1)
        def _():
            def zero_chunk(c, _):
                pltpu.sync_copy(zblk_hbm, acc_sh.at[pl.ds(zbase + c * 128, 128)])
                return _

            lax.fori_loop(0, 5, zero_chunk, None)

        @pl.when(sid == SC_SUBCORES - 1)
        def _():
            def zero_chunk(c, _):
                pltpu.sync_copy(zblk_hbm, acc_sh.at[pl.ds(zbase + c * 128, 128)])
                return _

            lax.fori_loop(0, 3, zero_chunk, None)
            pltpu.sync_copy(zblk_hbm.at[pl.ds(0, 16)],
                            acc_sh.at[pl.ds(zbase + 384, 16)])

        plsc.subcore_barrier()

        estart = sid * per_sc

        def add_chunk(c, _):
            off = estart + c * 128
            pltpu.sync_copy(obj_hbm.at[pl.ds(off, 128)], idx_v)

            @pl.when(cid == 0)
            def _():
                pltpu.sync_copy(msg_hbm.at[pl.ds(off, 128)], rows_v)

            @pl.when(cid == 1)
            def _():
                pltpu.sync_copy(sq_hbm.at[pl.ds(off, 128)], rows_v)

            pltpu.sync_copy(rows_v, acc_sh.at[idx_v], add=True)
            return _

        lax.fori_loop(0, nch1, add_chunk, None)
        if rem1:
            off = estart + nch1 * 128
            pltpu.sync_copy(obj_hbm.at[pl.ds(off, rem1)], idx_r)

            @pl.when(cid == 0)
            def _():
                pltpu.sync_copy(msg_hbm.at[pl.ds(off, rem1)], rows_r)

            @pl.when(cid == 1)
            def _():
                pltpu.sync_copy(sq_hbm.at[pl.ds(off, rem1)], rows_r)

            pltpu.sync_copy(rows_r, acc_sh.at[idx_r], add=True)

        plsc.subcore_barrier()

        def dump128(off):
            pltpu.sync_copy(acc_sh.at[pl.ds(off, 128)], rows_v)

            @pl.when(cid == 0)
            def _():
                pltpu.sync_copy(rows_v, ssum_hbm.at[pl.ds(off, 128)])

            @pl.when(cid == 1)
            def _():
                pltpu.sync_copy(rows_v, ssq_hbm.at[pl.ds(off, 128)])

        @pl.when(sid < SC_SUBCORES - 1)
        def _():
            def dump_chunk(c, _):
                dump128(zbase + c * 128)
                return _

            lax.fori_loop(0, 5, dump_chunk, None)

        @pl.when(sid == SC_SUBCORES - 1)
        def _():
            def dump_chunk(c, _):
                dump128(zbase + c * 128)
                return _

            lax.fori_loop(0, 3, dump_chunk, None)
            off = zbase + 384
            pltpu.sync_copy(acc_sh.at[pl.ds(off, 16)], rows_v.at[pl.ds(0, 16)])

            @pl.when(cid == 0)
            def _():
                pltpu.sync_copy(rows_v.at[pl.ds(0, 16)],
                                ssum_hbm.at[pl.ds(off, 16)])

            @pl.when(cid == 1)
            def _():
                pltpu.sync_copy(rows_v.at[pl.ds(0, 16)],
                                ssq_hbm.at[pl.ds(off, 16)])

    return k(msg, msgsq, obj, zblk)


def _smm_body(obj_ref, msg_ref, mx_ref, mn_ref, cnt_ref, *, ec):
    i = pl.program_id(0)

    @pl.when(i == 0)
    def _():
        mx_ref[...] = jnp.full(mx_ref.shape, -_SENT, jnp.float32)
        mn_ref[...] = jnp.full(mn_ref.shape, _SENT, jnp.float32)
        cnt_ref[...] = jnp.zeros(cnt_ref.shape, jnp.float32)

    def step(k, _):
        o = obj_ref[0, 0, k]
        row = msg_ref[pl.ds(k, 1), :]
        mx_ref[pl.ds(o, 1), :] = jnp.maximum(mx_ref[pl.ds(o, 1), :], row)
        mn_ref[pl.ds(o, 1), :] = jnp.minimum(mn_ref[pl.ds(o, 1), :], row)
        cnt_ref[pl.ds(o, 1), :] = cnt_ref[pl.ds(o, 1), :] + 1.0
        return _

    lax.fori_loop(0, ec, step, None)


def _seg_maxmin(messages, obj, n, ec=2000):
    e, d = messages.shape
    return pl.pallas_call(
        functools.partial(_smm_body, ec=ec),
        grid=(e // ec,),
        in_specs=[
            pl.BlockSpec((1, 1, ec), lambda i: (i, 0, 0),
                         memory_space=pltpu.SMEM),
            pl.BlockSpec((ec, d), lambda i: (i, 0)),
        ],
        out_specs=[pl.BlockSpec((n, d), lambda i: (0, 0)),
                   pl.BlockSpec((n, d), lambda i: (0, 0)),
                   pl.BlockSpec((n, 1), lambda i: (0, 0))],
        out_shape=[jax.ShapeDtypeStruct((n, d), jnp.float32),
                   jax.ShapeDtypeStruct((n, d), jnp.float32),
                   jax.ShapeDtypeStruct((n, 1), jnp.float32)],
    )(obj.reshape(e // ec, 1, ec), messages)


def _msg_body(hs_ref, rel_ref, wh_ref, bh_ref, relgi_ref, out_ref, outsq_ref):
    hs = hs_ref[...]
    gh = jnp.dot(hs, wh_ref[...], preferred_element_type=jnp.float32)
    gh = gh + bh_ref[...]
    rel = rel_ref[0, 0, :]
    nrel = relgi_ref.shape[0]
    oh = (rel[:, None] == lax.broadcasted_iota(jnp.int32, (rel.shape[0], nrel), 1)
          ).astype(jnp.float32)
    gi = jnp.dot(oh, relgi_ref[...], preferred_element_type=jnp.float32)
    d = hs.shape[1]
    i_r, i_z, i_n = gi[:, :d], gi[:, d:2 * d], gi[:, 2 * d:]
    h_r, h_z, h_n = gh[:, :d], gh[:, d:2 * d], gh[:, 2 * d:]
    r = jax.nn.sigmoid(i_r + h_r)
    z = jax.nn.sigmoid(i_z + h_z)
    n = jnp.tanh(i_n + r * h_n)
    msg = (1.0 - z) * n + z * hs
    out_ref[...] = msg
    outsq_ref[...] = msg * msg


def _compute_messages(hs_g, rel, gru_Wh, gru_bh, rel_gi, eb=2000):
    e = hs_g.shape[0]
    d = hs_g.shape[1]
    nrel = rel_gi.shape[0]
    nb = e // eb
    rel3 = rel.reshape(nb, 1, eb)
    return pl.pallas_call(
        _msg_body,
        grid=(nb,),
        in_specs=[
            pl.BlockSpec((eb, d), lambda i: (i, 0)),
            pl.BlockSpec((1, 1, eb), lambda i: (i, 0, 0)),
            pl.BlockSpec((d, 3 * d), lambda i: (0, 0)),
            pl.BlockSpec((1, 3 * d), lambda i: (0, 0)),
            pl.BlockSpec((nrel, 3 * d), lambda i: (0, 0)),
        ],
        out_specs=[pl.BlockSpec((eb, d), lambda i: (i, 0)),
                   pl.BlockSpec((eb, d), lambda i: (i, 0))],
        out_shape=[jax.ShapeDtypeStruct((e, d), jnp.float32),
                   jax.ShapeDtypeStruct((e, d), jnp.float32)],
    )(hs_g, rel3, gru_Wh, gru_bh.reshape(1, 3 * d), rel_gi)


def _pna_body(ssum_ref, sq_ref, mx_ref, mn_ref, deg_ref, cnt01_ref,
              hpn_ref, nodes_ref, pnaw_ref, pnab_ref,
              ht_ref, usum_ref, ucnt_ref):
    i = pl.program_id(0)
    deg = deg_ref[...]
    inv = 1.0 / deg
    cnt01 = cnt01_ref[...]
    mean = ssum_ref[...] * inv
    var = jnp.maximum(sq_ref[...] * inv - mean * mean, 0.0)
    std = jnp.sqrt(var + 1e-6)
    mx = jnp.where(cnt01 > 0.0, mx_ref[...], 0.0)
    mn = jnp.where(cnt01 > 0.0, mn_ref[...], 0.0)
    aggs = jnp.concatenate([mean, mx, mn, std], axis=1)
    logd = jnp.log(deg + 1.0)
    s_amp = logd * (1.0 / DELTA)
    s_att = DELTA / logd
    feat = jnp.concatenate(
        [hpn_ref[...], aggs, aggs * s_amp, aggs * s_att], axis=1)
    ht = jnp.dot(feat, pnaw_ref[...], preferred_element_type=jnp.float32)
    ht = ht + pnab_ref[...]
    ht_ref[...] = ht

    nodes = nodes_ref[...]
    n_batch = nodes[:, 0]
    n_id = nodes[:, 1]
    uw = (n_id < N_USER).astype(jnp.float32)
    oh = (n_batch[:, None] == lax.broadcasted_iota(
        jnp.int32, (nodes.shape[0], BATCH), 1)).astype(jnp.float32)
    hw = ht * uw[:, None]
    usum_part = lax.dot_general(oh, hw, (((0,), (0,)), ((), ())),
                                preferred_element_type=jnp.float32)
    uw2 = jnp.broadcast_to(uw[:, None], ht.shape)
    ucnt_part = lax.dot_general(oh, uw2, (((0,), (0,)), ((), ())),
                                preferred_element_type=jnp.float32)

    @pl.when(i == 0)
    def _():
        usum_ref[...] = jnp.zeros_like(usum_ref)
        ucnt_ref[...] = jnp.zeros_like(ucnt_ref)

    usum_ref[...] += usum_part
    ucnt_ref[...] += ucnt_part


def _pna(ssum, sq, mx, mn, deg, cnt01, hpn, nodes, pna_W, pna_b, nbk=1000):
    n, d = ssum.shape
    grid = (n // nbk,)
    blk = lambda w: pl.BlockSpec((nbk, w), lambda i: (i, 0))
    fix = lambda s: pl.BlockSpec(s, lambda i: tuple(0 for _ in s))
    return pl.pallas_call(
        _pna_body,
        grid=grid,
        in_specs=[blk(d), blk(d), blk(d), blk(d), blk(1), blk(1), blk(d),
                  blk(2), fix((13 * d, d)), fix((1, d))],
        out_specs=[blk(d), fix((BATCH, d)), fix((BATCH, d))],
        out_shape=[jax.ShapeDtypeStruct((n, d), jnp.float32),
                   jax.ShapeDtypeStruct((BATCH, d), jnp.float32),
                   jax.ShapeDtypeStruct((BATCH, d), jnp.float32)],
    )(ssum, sq, mx, mn, deg, cnt01, hpn, nodes, pna_W, pna_b.reshape(1, d))


def _prune_body(ht_ref, nodes_ref, huser_ref, w1_ref, b1_ref, w2_ref, b2_ref,
                hg_ref, alpha_ref):
    ht = ht_ref[...]
    nodes = nodes_ref[...]
    n_batch = nodes[:, 0]
    oh = (n_batch[:, None] == lax.broadcasted_iota(
        jnp.int32, (nodes.shape[0], BATCH), 1)).astype(jnp.float32)
    hu = jnp.dot(oh, huser_ref[...], preferred_element_type=jnp.float32)
    feat = jnp.concatenate([hu, ht], axis=1)
    l1 = jnp.maximum(
        jnp.dot(feat, w1_ref[...], preferred_element_type=jnp.float32)
        + b1_ref[...], 0.0)
    logit = jnp.dot(l1, w2_ref[...], preferred_element_type=jnp.float32)
    logit = logit + b2_ref[...]
    alpha = jax.nn.sigmoid(logit * (1.0 / TAU))
    alpha_ref[...] = alpha
    hg_ref[...] = alpha * ht


def _prune(ht, nodes, h_user, pr_W1, pr_b1, pr_W2, pr_b2, nbk=1000):
    n, d = ht.shape
    hid = pr_W1.shape[1]
    w2p = jnp.broadcast_to(pr_W2, (hid, d))
    b2p = jnp.broadcast_to(pr_b2.reshape(1, 1), (1, d))
    grid = (n // nbk,)
    blk = lambda w: pl.BlockSpec((nbk, w), lambda i: (i, 0))
    fix = lambda s: pl.BlockSpec(s, lambda i: tuple(0 for _ in s))
    return pl.pallas_call(
        _prune_body,
        grid=grid,
        in_specs=[blk(d), blk(2), fix((BATCH, d)), fix((2 * d, hid)),
                  fix((1, hid)), fix((hid, d)), fix((1, d))],
        out_specs=[blk(d), blk(d)],
        out_shape=[jax.ShapeDtypeStruct((n, d), jnp.float32),
                   jax.ShapeDtypeStruct((n, d), jnp.float32)],
    )(ht, nodes, h_user, pr_W1, pr_b1.reshape(1, hid), w2p, b2p)


def kernel(q_sub, q_rel, hidden, edges, nodes, id_layer, n_layer,
           old_nodes_new_idx, rel_table, gru_Wi, gru_Wh, gru_bi, gru_bh,
           pna_W, pna_b, pr_W1, pr_b1, pr_W2, pr_b2):
    n, d = hidden.shape
    e = edges.shape[0]

    sub = edges[:, 4]
    rel = edges[:, 2]
    obj = edges[:, 5]

    rel_gi = rel_table @ gru_Wi + gru_bi

    sub_c = jnp.asarray(sub, jnp.int32)
    obj_c = jnp.asarray(obj, jnp.int32)
    hs_g, hpn = _sc_gather_scatter(hidden, sub_c, old_nodes_new_idx)
    messages, messages_sq = _compute_messages(hs_g, rel, gru_Wh, gru_bh,
                                              rel_gi)

    ssum, sq = _sc_segment_sums(messages, messages_sq, obj_c)
    mx, mn, cnt2d = _seg_maxmin(messages, obj_c, n)
    cnt = cnt2d[:, 0]

    deg = jnp.maximum(cnt, 1.0).reshape(n, 1)
    cnt01 = (cnt > 0).astype(jnp.float32).reshape(n, 1)

    ht, usum, ucnt = _pna(ssum, sq, mx, mn, deg, cnt01, hpn, nodes,
                          pna_W, pna_b)
    h_user = usum / jnp.maximum(ucnt, 1.0)

    h_gated, alpha2 = _prune(ht, nodes, h_user, pr_W1, pr_b1, pr_W2, pr_b2)
    alpha = alpha2[:, 0]

    sampled_nodes_idx = jnp.ones((n,), dtype=bool)
    final_nodes = jnp.array([0], dtype=jnp.int32)
    return (h_gated, nodes, final_nodes, old_nodes_new_idx,
            sampled_nodes_idx, alpha, edges)
